# Initial kernel scaffold; baseline (speedup 1.0000x reference)
import jax
import jax.numpy as jnp
from jax.experimental import pallas as pl


def kernel(cls_pred, loc_pred, ctr_pred, locations):
    raise NotImplementedError("write your pallas kernel here")


# speedup vs baseline: 2.6070x; 2.6070x over previous
import functools

import jax
import jax.numpy as jnp
from jax.experimental import pallas as pl

NUM_CLASSES = 80
PRE_NMS_TOP_N = 1000
POST_NMS_TOP_N = 100
IOU_THR = 0.6
SCORE_THR = 0.05
CLASS_OFFSET = 4096.0

B = 2
K = 20000
KP = 20480
ROWS = 160
LANES = 128
BLK = 2000
KB = K // BLK


def _map_kernel(cls_ref, loc_ref, ctr_ref, locs_ref, score_ref, label_ref,
                box_ref):
  cls = cls_ref[0, 0]
  ctr = ctr_ref[0, 0]
  prob = jax.nn.sigmoid(cls) * jax.nn.sigmoid(ctr)
  m = jnp.max(prob, axis=1)
  lane = jax.lax.broadcasted_iota(jnp.int32, prob.shape, 1)
  amax = jnp.min(jnp.where(prob == m[:, None], lane, NUM_CLASSES), axis=1)
  score_ref[0, 0, 0, :] = m
  label_ref[0, 0, 0, :] = amax.astype(jnp.float32)
  off = loc_ref[0, 0]
  lx = locs_ref[0, :, 0]
  ly = locs_ref[0, :, 1]
  x1 = lx - off[:, 0]
  y1 = ly - off[:, 1]
  x2 = lx + off[:, 2]
  y2 = ly + off[:, 3]
  box_ref[0, :, 0, 0, :] = jnp.stack([x1, y1, x2, y2], axis=0)


def _select_mask(scores):
  u = jax.lax.bitcast_convert_type(scores, jnp.int32)

  def bit_body(i, lo):
    t = lo + jnp.left_shift(jnp.int32(1), 30 - i)
    c = jnp.sum((u >= t).astype(jnp.int32))
    return jnp.where(c >= PRE_NMS_TOP_N, t, lo)

  thr = jax.lax.fori_loop(0, 31, bit_body, jnp.int32(0))
  gt = (u > thr)
  n_gt = jnp.sum(gt.astype(jnp.int32))
  need = PRE_NMS_TOP_N - n_gt
  tie = (u == thr).astype(jnp.float32)
  r_iota = jax.lax.broadcasted_iota(jnp.int32, (ROWS, ROWS), 0)
  c_iota = jax.lax.broadcasted_iota(jnp.int32, (ROWS, ROWS), 1)
  strict_l = (c_iota < r_iota).astype(jnp.float32)
  r2 = jax.lax.broadcasted_iota(jnp.int32, (LANES, LANES), 0)
  c2 = jax.lax.broadcasted_iota(jnp.int32, (LANES, LANES), 1)
  strict_u = (r2 < c2).astype(jnp.float32)
  row_cnt = jnp.dot(tie, jnp.ones((LANES, 1), jnp.float32),
                    preferred_element_type=jnp.float32)
  row_pref = jnp.dot(strict_l, row_cnt,
                     preferred_element_type=jnp.float32)
  lane_pref = jnp.dot(tie, strict_u,
                      preferred_element_type=jnp.float32)
  rank = row_pref + lane_pref
  sel = gt | ((u == thr) & (rank < need.astype(jnp.float32)))
  return sel


def _nms_kernel(score_ref, label_ref, box_ref, out_ref):
  scores = score_ref[0]
  labels = label_ref[0]
  sel = _select_mask(scores)
  s0 = jnp.where(sel & (scores > SCORE_THR), scores, -1e9)

  x1 = box_ref[0, 0]
  y1 = box_ref[0, 1]
  x2 = box_ref[0, 2]
  y2 = box_ref[0, 3]
  offs = labels * CLASS_OFFSET
  x1o = x1 + offs
  y1o = y1 + offs
  x2o = x2 + offs
  y2o = y2 + offs
  areas = jnp.maximum(x2o - x1o, 0.0) * jnp.maximum(y2o - y1o, 0.0)
  lin = (jax.lax.broadcasted_iota(jnp.int32, (ROWS, LANES), 0) * LANES
         + jax.lax.broadcasted_iota(jnp.int32, (ROWS, LANES), 1))
  lane_i = jax.lax.broadcasted_iota(jnp.int32, (1, LANES), 1)

  def body(i, s):
    sc = jnp.max(s)
    il = jnp.min(jnp.where(s == sc, lin, jnp.int32(2**30)))
    r = il // LANES
    c = il % LANES

    def pick(row):
      return jnp.sum(jnp.where(lane_i == c, row, 0.0))

    bx1 = pick(box_ref[0, 0, pl.ds(r, 1), :])
    by1 = pick(box_ref[0, 1, pl.ds(r, 1), :])
    bx2 = pick(box_ref[0, 2, pl.ds(r, 1), :])
    by2 = pick(box_ref[0, 3, pl.ds(r, 1), :])
    lab = pick(label_ref[0, pl.ds(r, 1), :])
    det = jnp.concatenate([
        jnp.full((1, 1), v, jnp.float32)
        for v in (bx1, by1, bx2, by2, sc, lab)
    ], axis=1)
    det = jnp.where(sc > -1e8, det, jnp.zeros_like(det))
    out_ref[0, pl.ds(i, 1), :] = det
    o = lab * CLASS_OFFSET
    b0 = bx1 + o
    b1 = by1 + o
    b2 = bx2 + o
    b3 = by2 + o
    xx1 = jnp.maximum(b0, x1o)
    yy1 = jnp.maximum(b1, y1o)
    xx2 = jnp.minimum(b2, x2o)
    yy2 = jnp.minimum(b3, y2o)
    inter = jnp.maximum(xx2 - xx1, 0.0) * jnp.maximum(yy2 - yy1, 0.0)
    a = jnp.maximum(b2 - b0, 0.0) * jnp.maximum(b3 - b1, 0.0)
    iou = inter / (a + areas - inter + 1e-9)
    s = jnp.where(iou > IOU_THR, -1e9, s)
    s = jnp.where(lin == il, -1e9, s)
    return s

  jax.lax.fori_loop(0, POST_NMS_TOP_N, body, s0)


@jax.jit
def kernel(cls_pred, loc_pred, ctr_pred, locations):
  cls4 = cls_pred.reshape(B, KB, BLK, NUM_CLASSES)
  loc4 = loc_pred.reshape(B, KB, BLK, 4)
  ctr4 = ctr_pred.reshape(B, KB, BLK, 1)
  locs3 = locations.reshape(KB, BLK, 2)

  scores, labels, boxes = pl.pallas_call(
      _map_kernel,
      grid=(B, KB),
      in_specs=[
          pl.BlockSpec((1, 1, BLK, NUM_CLASSES), lambda b, k: (b, k, 0, 0)),
          pl.BlockSpec((1, 1, BLK, 4), lambda b, k: (b, k, 0, 0)),
          pl.BlockSpec((1, 1, BLK, 1), lambda b, k: (b, k, 0, 0)),
          pl.BlockSpec((1, BLK, 2), lambda b, k: (k, 0, 0)),
      ],
      out_specs=[
          pl.BlockSpec((1, 1, 1, BLK), lambda b, k: (b, k, 0, 0)),
          pl.BlockSpec((1, 1, 1, BLK), lambda b, k: (b, k, 0, 0)),
          pl.BlockSpec((1, 4, 1, 1, BLK), lambda b, k: (b, 0, k, 0, 0)),
      ],
      out_shape=[
          jax.ShapeDtypeStruct((B, KB, 1, BLK), jnp.float32),
          jax.ShapeDtypeStruct((B, KB, 1, BLK), jnp.float32),
          jax.ShapeDtypeStruct((B, 4, KB, 1, BLK), jnp.float32),
      ],
  )(cls4, loc4, ctr4, locs3)

  pad = KP - K
  scores_p = jnp.pad(scores.reshape(B, K), ((0, 0), (0, pad)))
  labels_p = jnp.pad(labels.reshape(B, K), ((0, 0), (0, pad)))
  boxes_p = jnp.pad(boxes.reshape(B, 4, K), ((0, 0), (0, 0), (0, pad)))

  dets = pl.pallas_call(
      _nms_kernel,
      grid=(B,),
      in_specs=[
          pl.BlockSpec((1, ROWS, LANES), lambda b: (b, 0, 0)),
          pl.BlockSpec((1, ROWS, LANES), lambda b: (b, 0, 0)),
          pl.BlockSpec((1, 4, ROWS, LANES), lambda b: (b, 0, 0, 0)),
      ],
      out_specs=pl.BlockSpec((1, POST_NMS_TOP_N, 6), lambda b: (b, 0, 0)),
      out_shape=jax.ShapeDtypeStruct((B, POST_NMS_TOP_N, 6), jnp.float32),
  )(scores_p.reshape(B, ROWS, LANES), labels_p.reshape(B, ROWS, LANES),
    boxes_p.reshape(B, 4, ROWS, LANES))
  return dets


# speedup vs baseline: 2.6932x; 1.0331x over previous
import functools

import jax
import jax.numpy as jnp
from jax.experimental import pallas as pl

NUM_CLASSES = 80
PRE_NMS_TOP_N = 1000
POST_NMS_TOP_N = 100
IOU_THR = 0.6
SCORE_THR = 0.05
CLASS_OFFSET = 4096.0

B = 2
K = 20000
KP = 20480
ROWS = 160
LANES = 128
BLK = 2000
BLKP = 2048
KB = K // BLK


def _map_kernel(cls_ref, loc_ref, ctr_ref, locs_ref, score_ref, label_ref,
                box_ref):
  cls = cls_ref[0, 0]
  ctr = ctr_ref[0, 0]
  prob = jax.nn.sigmoid(cls) * jax.nn.sigmoid(ctr)
  m = jnp.max(prob, axis=1)
  lane = jax.lax.broadcasted_iota(jnp.int32, prob.shape, 1)
  amax = jnp.min(jnp.where(prob == m[:, None], lane, NUM_CLASSES), axis=1)
  zpad = jnp.zeros((BLKP - BLK,), jnp.float32)
  score_ref[0, 0, 0, :] = jnp.concatenate([m, zpad])
  label_ref[0, 0, 0, :] = jnp.concatenate([amax.astype(jnp.float32), zpad])
  off = loc_ref[0, 0]
  lx = locs_ref[0, :, 0]
  ly = locs_ref[0, :, 1]
  x1 = lx - off[:, 0]
  y1 = ly - off[:, 1]
  x2 = lx + off[:, 2]
  y2 = ly + off[:, 3]
  bx = jnp.stack([x1, y1, x2, y2], axis=0)
  box_ref[0, :, 0, 0, :] = jnp.pad(bx, ((0, 0), (0, BLKP - BLK)))


def _select_mask(scores):
  u = jax.lax.bitcast_convert_type(scores, jnp.int32)

  def bit_body(i, lo):
    t = lo + jnp.left_shift(jnp.int32(1), 30 - i)
    c = jnp.sum((u >= t).astype(jnp.int32))
    return jnp.where(c >= PRE_NMS_TOP_N, t, lo)

  thr = jax.lax.fori_loop(0, 31, bit_body, jnp.int32(0))
  gt = (u > thr)
  n_gt = jnp.sum(gt.astype(jnp.int32))
  need = PRE_NMS_TOP_N - n_gt
  tie = (u == thr).astype(jnp.float32)
  r_iota = jax.lax.broadcasted_iota(jnp.int32, (ROWS, ROWS), 0)
  c_iota = jax.lax.broadcasted_iota(jnp.int32, (ROWS, ROWS), 1)
  strict_l = (c_iota < r_iota).astype(jnp.float32)
  r2 = jax.lax.broadcasted_iota(jnp.int32, (LANES, LANES), 0)
  c2 = jax.lax.broadcasted_iota(jnp.int32, (LANES, LANES), 1)
  strict_u = (r2 < c2).astype(jnp.float32)
  row_cnt = jnp.dot(tie, jnp.ones((LANES, 1), jnp.float32),
                    preferred_element_type=jnp.float32)
  row_pref = jnp.dot(strict_l, row_cnt,
                     preferred_element_type=jnp.float32)
  lane_pref = jnp.dot(tie, strict_u,
                      preferred_element_type=jnp.float32)
  rank = row_pref + lane_pref
  sel = gt | ((u == thr) & (rank < need.astype(jnp.float32)))
  return sel


def _nms_kernel(score_ref, label_ref, box_ref, out_ref):
  scores = score_ref[0]
  labels = label_ref[0]
  sel = _select_mask(scores)
  s0 = jnp.where(sel & (scores > SCORE_THR), scores, -1e9)

  x1 = box_ref[0, 0]
  y1 = box_ref[0, 1]
  x2 = box_ref[0, 2]
  y2 = box_ref[0, 3]
  offs = labels * CLASS_OFFSET
  x1o = x1 + offs
  y1o = y1 + offs
  x2o = x2 + offs
  y2o = y2 + offs
  areas = jnp.maximum(x2o - x1o, 0.0) * jnp.maximum(y2o - y1o, 0.0)
  lin = (jax.lax.broadcasted_iota(jnp.int32, (ROWS, LANES), 0) * LANES
         + jax.lax.broadcasted_iota(jnp.int32, (ROWS, LANES), 1))
  lane_i = jax.lax.broadcasted_iota(jnp.int32, (1, LANES), 1)

  def body(i, s):
    sc = jnp.max(s)
    il = jnp.min(jnp.where(s == sc, lin, jnp.int32(2**30)))
    r = il // LANES
    c = il % LANES

    def pick(row):
      return jnp.sum(jnp.where(lane_i == c, row, 0.0))

    bx1 = pick(box_ref[0, 0, pl.ds(r, 1), :])
    by1 = pick(box_ref[0, 1, pl.ds(r, 1), :])
    bx2 = pick(box_ref[0, 2, pl.ds(r, 1), :])
    by2 = pick(box_ref[0, 3, pl.ds(r, 1), :])
    lab = pick(label_ref[0, pl.ds(r, 1), :])
    det = jnp.concatenate([
        jnp.full((1, 1), v, jnp.float32)
        for v in (bx1, by1, bx2, by2, sc, lab)
    ], axis=1)
    det = jnp.where(sc > -1e8, det, jnp.zeros_like(det))
    out_ref[0, pl.ds(i, 1), :] = det
    o = lab * CLASS_OFFSET
    b0 = bx1 + o
    b1 = by1 + o
    b2 = bx2 + o
    b3 = by2 + o
    xx1 = jnp.maximum(b0, x1o)
    yy1 = jnp.maximum(b1, y1o)
    xx2 = jnp.minimum(b2, x2o)
    yy2 = jnp.minimum(b3, y2o)
    inter = jnp.maximum(xx2 - xx1, 0.0) * jnp.maximum(yy2 - yy1, 0.0)
    a = jnp.maximum(b2 - b0, 0.0) * jnp.maximum(b3 - b1, 0.0)
    iou = inter / (a + areas - inter + 1e-9)
    s = jnp.where(iou > IOU_THR, -1e9, s)
    s = jnp.where(lin == il, -1e9, s)
    return s

  jax.lax.fori_loop(0, POST_NMS_TOP_N, body, s0)


@jax.jit
def kernel(cls_pred, loc_pred, ctr_pred, locations):
  cls4 = cls_pred.reshape(B, KB, BLK, NUM_CLASSES)
  loc4 = loc_pred.reshape(B, KB, BLK, 4)
  ctr4 = ctr_pred.reshape(B, KB, BLK, 1)
  locs3 = locations.reshape(KB, BLK, 2)

  scores, labels, boxes = pl.pallas_call(
      _map_kernel,
      grid=(B, KB),
      in_specs=[
          pl.BlockSpec((1, 1, BLK, NUM_CLASSES), lambda b, k: (b, k, 0, 0)),
          pl.BlockSpec((1, 1, BLK, 4), lambda b, k: (b, k, 0, 0)),
          pl.BlockSpec((1, 1, BLK, 1), lambda b, k: (b, k, 0, 0)),
          pl.BlockSpec((1, BLK, 2), lambda b, k: (k, 0, 0)),
      ],
      out_specs=[
          pl.BlockSpec((1, 1, 1, BLKP), lambda b, k: (b, k, 0, 0)),
          pl.BlockSpec((1, 1, 1, BLKP), lambda b, k: (b, k, 0, 0)),
          pl.BlockSpec((1, 4, 1, 1, BLKP), lambda b, k: (b, 0, k, 0, 0)),
      ],
      out_shape=[
          jax.ShapeDtypeStruct((B, KB, 1, BLKP), jnp.float32),
          jax.ShapeDtypeStruct((B, KB, 1, BLKP), jnp.float32),
          jax.ShapeDtypeStruct((B, 4, KB, 1, BLKP), jnp.float32),
      ],
  )(cls4, loc4, ctr4, locs3)

  scores_p = scores.reshape(B, KP)
  labels_p = labels.reshape(B, KP)
  boxes_p = boxes.reshape(B, 4, KP)

  dets = pl.pallas_call(
      _nms_kernel,
      grid=(B,),
      in_specs=[
          pl.BlockSpec((1, ROWS, LANES), lambda b: (b, 0, 0)),
          pl.BlockSpec((1, ROWS, LANES), lambda b: (b, 0, 0)),
          pl.BlockSpec((1, 4, ROWS, LANES), lambda b: (b, 0, 0, 0)),
      ],
      out_specs=pl.BlockSpec((1, POST_NMS_TOP_N, 6), lambda b: (b, 0, 0)),
      out_shape=jax.ShapeDtypeStruct((B, POST_NMS_TOP_N, 6), jnp.float32),
  )(scores_p.reshape(B, ROWS, LANES), labels_p.reshape(B, ROWS, LANES),
    boxes_p.reshape(B, 4, ROWS, LANES))
  return dets
